# baseline (speedup 1.0000x reference)
import functools

import jax
import jax.numpy as jnp
from jax import lax
from jax.experimental import pallas as pl
from jax.experimental.pallas import tpu as pltpu
from jax.experimental.pallas import tpu_sc as plsc

B = 16384
L = 50
D = 64
LP = 64
RING = 4


def _make_kernel(n_workers):
    bags_per_w = B // n_workers
    mesh = plsc.VectorSubcoreMesh(core_axis_name="c", subcore_axis_name="s")

    @functools.partial(
        pl.kernel,
        mesh=mesh,
        out_type=jax.ShapeDtypeStruct((B, D), jnp.float32),
        compiler_params=pltpu.CompilerParams(
            use_tc_tiling_on_sc=False, needs_layout_passes=False
        ),
        scratch_types=[
            pltpu.VMEM((LP, bags_per_w), jnp.int32),
            pltpu.VMEM((bags_per_w * LP,), jnp.int32),
            pltpu.VMEM((RING, L, D), jnp.float32),
            pltpu.VMEM((bags_per_w, D), jnp.float32),
        ]
        + [pltpu.SemaphoreType.DMA] * RING,
    )
    def embag(idxt_hbm, w_hbm, out_hbm, stage_v, lists_v, rows_v, out_v,
              *sems):
        n_cores = lax.axis_size("c")
        wid = lax.axis_index("s") * n_cores + lax.axis_index("c")
        base = wid * bags_per_w

        pltpu.sync_copy(
            idxt_hbm.at[:, pl.ds(base, bags_per_w)],
            stage_v.at[pl.ds(0, L), :],
        )

        iota = lax.iota(jnp.int32, 16)
        jvecs = [iota + 16 * g for g in range(LP // 16)]

        def build_body(b, _):
            bvec = iota * 0 + b
            for g in range(LP // 16):
                vals = plsc.load_gather(stage_v, [jvecs[g], bvec])
                lists_v[pl.ds(b * LP + 16 * g, 16)] = vals
            return ()

        lax.fori_loop(0, bags_per_w, build_body, ())

        for b in range(RING):
            pltpu.async_copy(
                w_hbm.at[lists_v.at[pl.ds(b * LP, L)]], rows_v.at[b], sems[b]
            )

        def group_body(p, _):
            for b in range(RING):
                c = p * RING + b
                pltpu.make_async_copy(
                    w_hbm.at[lists_v.at[pl.ds(c * LP, L)]],
                    rows_v.at[b], sems[b],
                ).wait()
                acc = [rows_v[b, 0, pl.ds(d * 16, 16)] for d in range(D // 16)]
                for r in range(1, L):
                    for d in range(D // 16):
                        acc[d] = acc[d] + rows_v[b, r, pl.ds(d * 16, 16)]
                for d in range(D // 16):
                    out_v[c, pl.ds(d * 16, 16)] = acc[d]

                @pl.when(c + RING < bags_per_w)
                def _():
                    pltpu.async_copy(
                        w_hbm.at[lists_v.at[pl.ds((c + RING) * LP, L)]],
                        rows_v.at[b], sems[b],
                    )

            return ()

        lax.fori_loop(0, bags_per_w // RING, group_body, ())

        pltpu.sync_copy(
            out_v, out_hbm.at[pl.ds(wid * bags_per_w, bags_per_w), :]
        )

    return embag


@jax.jit
def kernel(indices, W):
    info = plsc.get_sparse_core_info()
    n_workers = info.num_cores * info.num_subcores
    idxt = jnp.transpose(indices.astype(jnp.int32))
    return _make_kernel(n_workers)(idxt, W)


# speedup vs baseline: 1.0233x; 1.0233x over previous
import functools

import jax
import jax.numpy as jnp
from jax import lax
from jax.experimental import pallas as pl
from jax.experimental.pallas import tpu as pltpu
from jax.experimental.pallas import tpu_sc as plsc

B = 16384
L = 50
D = 64
RING = 4


def _make_kernel(n_workers):
    bags_per_w = B // n_workers
    mesh = plsc.VectorSubcoreMesh(core_axis_name="c", subcore_axis_name="s")

    @functools.partial(
        pl.kernel,
        mesh=mesh,
        out_type=jax.ShapeDtypeStruct((B, D), jnp.float32),
        compiler_params=pltpu.CompilerParams(use_tc_tiling_on_sc=False),
        scratch_types=[
            pltpu.VMEM((bags_per_w * 128,), jnp.int32),
            pltpu.VMEM((RING, L, D), jnp.float32),
            pltpu.VMEM((bags_per_w, D), jnp.float32),
        ]
        + [pltpu.SemaphoreType.DMA] * RING,
    )
    def embag(idx_hbm, w_hbm, out_hbm, idx_v, rows_v, out_v, *sems):
        n_cores = lax.axis_size("c")
        wid = lax.axis_index("s") * n_cores + lax.axis_index("c")

        pltpu.sync_copy(
            idx_hbm.at[pl.ds(wid * bags_per_w * 128, bags_per_w * 128)], idx_v
        )

        for b in range(RING):
            pltpu.async_copy(
                w_hbm.at[idx_v.at[pl.ds(b * 128, L)]], rows_v.at[b], sems[b]
            )

        def group_body(p, _):
            for b in range(RING):
                c = p * RING + b
                pltpu.make_async_copy(
                    w_hbm.at[idx_v.at[pl.ds(c * 128, L)]],
                    rows_v.at[b], sems[b],
                ).wait()
                acc = [rows_v[b, 0, pl.ds(d * 16, 16)] for d in range(D // 16)]
                for r in range(1, L):
                    for d in range(D // 16):
                        acc[d] = acc[d] + rows_v[b, r, pl.ds(d * 16, 16)]
                for d in range(D // 16):
                    out_v[c, pl.ds(d * 16, 16)] = acc[d]

                @pl.when(c + RING < bags_per_w)
                def _():
                    pltpu.async_copy(
                        w_hbm.at[idx_v.at[pl.ds((c + RING) * 128, L)]],
                        rows_v.at[b], sems[b],
                    )

            return ()

        lax.fori_loop(0, bags_per_w // RING, group_body, ())

        pltpu.sync_copy(
            out_v, out_hbm.at[pl.ds(wid * bags_per_w, bags_per_w), :]
        )

    return embag


@jax.jit
def kernel(indices, W):
    info = plsc.get_sparse_core_info()
    n_workers = info.num_cores * info.num_subcores
    idxp = jnp.pad(indices.astype(jnp.int32), ((0, 0), (0, 128 - L)))
    wt = lax.optimization_barrier(jnp.transpose(W))
    w_lin = jnp.transpose(wt)
    return _make_kernel(n_workers)(jnp.reshape(idxp, (-1,)), w_lin)
